# baseline (speedup 1.0000x reference)
import functools

import jax
import jax.numpy as jnp
from jax import lax
from jax.experimental import pallas as pl
from jax.experimental.pallas import tpu as pltpu
from jax.experimental.pallas import tpu_sc as plsc


def _sc_gather(table, idx):
    V, D = table.shape
    (B,) = idx.shape
    info = plsc.get_sparse_core_info()
    NW = info.num_cores * info.num_subcores
    b_per_w = B // NW
    C = 64
    n_chunks = b_per_w // C
    mesh = plsc.VectorSubcoreMesh(core_axis_name="c", subcore_axis_name="s")

    @functools.partial(
        pl.kernel,
        mesh=mesh,
        out_type=jax.ShapeDtypeStruct((B, D), jnp.float32),
        scratch_types=[
            pltpu.VMEM((b_per_w,), jnp.int32),
            pltpu.VMEM((C, D), jnp.float32),
            pltpu.SemaphoreType.DMA,
        ],
    )
    def k(table_hbm, idx_hbm, out_hbm, idx_v, rows_v, sem):
        wid = lax.axis_index("s") * info.num_cores + lax.axis_index("c")
        base = wid * b_per_w
        pltpu.sync_copy(idx_hbm.at[pl.ds(base, b_per_w)], idx_v)
        for c in range(n_chunks):
            pltpu.async_copy(
                table_hbm.at[idx_v.at[pl.ds(c * C, C)]], rows_v, sem
            ).wait()
            pltpu.sync_copy(rows_v, out_hbm.at[pl.ds(base + c * C, C)])

    return k(table, idx)


def _tc_compute(g, u, interpret=False):
    B, A = g.shape
    R = 256
    grid = B // R

    def body(g_ref, u_ref, act_ref, sp_ref, lp_ref):
        p = g_ref[...]
        e = -jnp.log(u_ref[...])
        r = p / e
        a = jnp.argmax(r, axis=-1)
        s = jnp.sum(p, axis=-1)
        cols = lax.broadcasted_iota(jnp.int32, p.shape, 1)
        v = jnp.sum(jnp.where(cols == a[:, None], p, 0.0), axis=-1)
        ratio = v / s
        act_ref[...] = a[:, None]
        sp_ref[...] = ratio[:, None]
        lp_ref[...] = jnp.log(ratio)[:, None]

    acts, sps, lps = pl.pallas_call(
        body,
        grid=(grid,),
        in_specs=[
            pl.BlockSpec((R, A), lambda i: (i, 0)),
            pl.BlockSpec((R, A), lambda i: (i, 0)),
        ],
        out_specs=[
            pl.BlockSpec((R, 1), lambda i: (i, 0)),
            pl.BlockSpec((R, 1), lambda i: (i, 0)),
            pl.BlockSpec((R, 1), lambda i: (i, 0)),
        ],
        out_shape=[
            jax.ShapeDtypeStruct((B, 1), jnp.int32),
            jax.ShapeDtypeStruct((B, 1), jnp.float32),
            jax.ShapeDtypeStruct((B, 1), jnp.float32),
        ],
        interpret=interpret,
    )(g, u)
    return acts[:, 0], sps[:, 0], lps[:, 0]


def kernel(probs_table, state_idx, u):
    (B,) = state_idx.shape
    N = 4
    S = B // N
    gs = [_sc_gather(probs_table, state_idx[k * S : (k + 1) * S]) for k in range(N)]
    outs = [_tc_compute(gs[k], u[k * S : (k + 1) * S]) for k in range(N)]
    return tuple(jnp.concatenate([o[i] for o in outs]) for i in range(3))


# speedup vs baseline: 1.3997x; 1.3997x over previous
import functools

import jax
import jax.numpy as jnp
from jax import lax
from jax.experimental import pallas as pl
from jax.experimental.pallas import tpu as pltpu
from jax.experimental.pallas import tpu_sc as plsc


def _sc_gather(table, idx):
    V, D = table.shape
    (B,) = idx.shape
    info = plsc.get_sparse_core_info()
    NW = info.num_cores * info.num_subcores
    b_per_w = B // NW
    C = 32
    n_chunks = b_per_w // C
    mesh = plsc.VectorSubcoreMesh(core_axis_name="c", subcore_axis_name="s")

    @functools.partial(
        pl.kernel,
        mesh=mesh,
        out_type=jax.ShapeDtypeStruct((B, D), jnp.float32),
        scratch_types=[
            pltpu.VMEM((b_per_w,), jnp.int32),
            pltpu.VMEM((C, D), jnp.float32),
            pltpu.VMEM((C, D), jnp.float32),
            pltpu.SemaphoreType.DMA,
            pltpu.SemaphoreType.DMA,
            pltpu.SemaphoreType.DMA,
            pltpu.SemaphoreType.DMA,
        ],
    )
    def k(table_hbm, idx_hbm, out_hbm, idx_v, buf0, buf1, g0, g1, w0, w1):
        wid = lax.axis_index("s") * info.num_cores + lax.axis_index("c")
        base = wid * b_per_w
        bufs, gsems, wsems = (buf0, buf1), (g0, g1), (w0, w1)
        pltpu.sync_copy(idx_hbm.at[pl.ds(base, b_per_w)], idx_v)
        pending_w = [None, None]
        for c in range(n_chunks):
            b = c % 2
            if pending_w[b] is not None:
                pending_w[b].wait()
            g = pltpu.async_copy(
                table_hbm.at[idx_v.at[pl.ds(c * C, C)]], bufs[b], gsems[b]
            )
            g.wait()
            w = pltpu.async_copy(
                bufs[b], out_hbm.at[pl.ds(base + c * C, C)], wsems[b]
            )
            pending_w[b] = w
        for b in range(2):
            if pending_w[b] is not None:
                pending_w[b].wait()

    return k(table, idx)


def _tc_compute(g, u, interpret=False):
    B, A = g.shape
    R = 512
    grid = B // R

    def body(g_ref, u_ref, act_ref, sp_ref, lp_ref):
        p = g_ref[...]
        e = -jnp.log(u_ref[...])
        r = p / e
        a = jnp.argmax(r, axis=-1)
        s = jnp.sum(p, axis=-1)
        cols = lax.broadcasted_iota(jnp.int32, p.shape, 1)
        v = jnp.sum(jnp.where(cols == a[:, None], p, 0.0), axis=-1)
        ratio = v / s
        act_ref[...] = a[:, None]
        sp_ref[...] = ratio[:, None]
        lp_ref[...] = jnp.log(ratio)[:, None]

    acts, sps, lps = pl.pallas_call(
        body,
        grid=(grid,),
        in_specs=[
            pl.BlockSpec((R, A), lambda i: (i, 0)),
            pl.BlockSpec((R, A), lambda i: (i, 0)),
        ],
        out_specs=[
            pl.BlockSpec((R, 1), lambda i: (i, 0)),
            pl.BlockSpec((R, 1), lambda i: (i, 0)),
            pl.BlockSpec((R, 1), lambda i: (i, 0)),
        ],
        out_shape=[
            jax.ShapeDtypeStruct((B, 1), jnp.int32),
            jax.ShapeDtypeStruct((B, 1), jnp.float32),
            jax.ShapeDtypeStruct((B, 1), jnp.float32),
        ],
        interpret=interpret,
    )(g, u)
    return acts[:, 0], sps[:, 0], lps[:, 0]


def kernel(probs_table, state_idx, u):
    g = _sc_gather(probs_table, state_idx)
    return _tc_compute(g, u)
